# Initial kernel scaffold; baseline (speedup 1.0000x reference)
import jax
import jax.numpy as jnp
from jax.experimental import pallas as pl


def kernel(feat_rgb, rot, shift_u, shift_v):
    raise NotImplementedError("write your pallas kernel here")


# speedup vs baseline: 6.4195x; 6.4195x over previous
import functools
import math

import jax
import jax.numpy as jnp
from jax import lax
from jax.experimental import pallas as pl
from jax.experimental.pallas import tpu as pltpu
from jax.experimental.pallas import tpu_sc as plsc

BEV = 256
IH = 512
IW = 512
HALF = 256
C = 96
B = 2
NPIX = BEV * BEV
NROWS = (IH - HALF) * IW

NC = 2
NS = 16
NW = NC * NS
PER_W = B * NPIX // NW
P = 128
NCHUNK = PER_W // P

_TWO_PI = 2.0 * math.pi


def _coord_body(rot_ref, su_ref, sv_ref, idx_ref, w_ref):
    b = pl.program_id(0)
    i = lax.broadcasted_iota(jnp.int32, (BEV, BEV), 0).astype(jnp.float32)
    j = lax.broadcasted_iota(jnp.int32, (BEV, BEV), 1).astype(jnp.float32)
    cv = BEV / 2 - 0.5 + sv_ref[b]
    cu = BEV / 2 - 0.5 + su_ref[b]
    dy = i - cv
    dx = j - cu
    radius = jnp.sqrt(dy * dy + dx * dx)
    theta = jnp.arctan2(dy, dx)
    theta = (-math.pi / 2 + theta % _TWO_PI) % _TWO_PI
    theta = (theta + rot_ref[b] * _TWO_PI) % _TWO_PI
    u = theta / _TWO_PI * IW
    phi = jnp.arctan2(radius, jnp.float32(-2.0))
    v = phi / math.pi * IH

    fx = jnp.floor(u)
    fy = jnp.floor(v)
    ix0 = jnp.clip(fx, 0.0, IW - 1)
    ix1 = jnp.clip(fx + 1.0, 0.0, IW - 1)
    iy0 = jnp.clip(fy, 0.0, IH - 1)
    iy1 = jnp.clip(fy + 1.0, 0.0, IH - 1)

    w_ref[0, 0] = (ix1 - u) * (iy1 - v)
    w_ref[1, 0] = (u - ix0) * (iy1 - v)
    w_ref[2, 0] = (ix1 - u) * (v - iy0)
    w_ref[3, 0] = (u - ix0) * (v - iy0)

    base = b * NROWS
    ix0i = ix0.astype(jnp.int32)
    ix1i = ix1.astype(jnp.int32)
    ry0 = jnp.clip(iy0.astype(jnp.int32) - HALF, 0, IH - HALF - 1) * IW
    ry1 = jnp.clip(iy1.astype(jnp.int32) - HALF, 0, IH - HALF - 1) * IW
    idx_ref[0, 0] = base + ry0 + ix0i
    idx_ref[1, 0] = base + ry0 + ix1i
    idx_ref[2, 0] = base + ry1 + ix0i
    idx_ref[3, 0] = base + ry1 + ix1i


def _compute_coords(rot, shift_u, shift_v):
    return pl.pallas_call(
        _coord_body,
        grid=(B,),
        in_specs=[
            pl.BlockSpec(memory_space=pltpu.SMEM),
            pl.BlockSpec(memory_space=pltpu.SMEM),
            pl.BlockSpec(memory_space=pltpu.SMEM),
        ],
        out_specs=[
            pl.BlockSpec((4, 1, BEV, BEV), lambda b: (0, b, 0, 0)),
            pl.BlockSpec((4, 1, BEV, BEV), lambda b: (0, b, 0, 0)),
        ],
        out_shape=[
            jax.ShapeDtypeStruct((4, B, BEV, BEV), jnp.int32),
            jax.ShapeDtypeStruct((4, B, BEV, BEV), jnp.float32),
        ],
    )(rot, shift_u, shift_v)


def _sc_body(img_hbm, idx_hbm, w_hbm, out_hbm, idx_v, w_v, rows_v, out_v, sem):
    wid = lax.axis_index("s") * NC + lax.axis_index("c")

    def chunk_body(t, carry):
        off = wid * PER_W + t * P
        pltpu.sync_copy(idx_hbm.at[:, pl.ds(off, P)], idx_v)
        pltpu.sync_copy(w_hbm.at[:, pl.ds(off, P)], w_v)
        cps = [
            pltpu.async_copy(img_hbm.at[idx_v.at[k]], rows_v.at[k], sem)
            for k in range(4)
        ]
        for cp in cps:
            cp.wait()

        def grp_body(g, c2):
            p0 = g * 16
            wvec = [w_v[k, pl.ds(p0, 16)] for k in range(4)]
            for pi in range(16):
                p = p0 + pi
                wk = [jnp.full((16,), wvec[k][pi], jnp.float32)
                      for k in range(4)]
                for vi in range(C // 16):
                    sl = pl.ds(vi * 16, 16)
                    out_v[p, sl] = (rows_v[0, p, sl] * wk[0]
                                    + rows_v[1, p, sl] * wk[1]
                                    + rows_v[2, p, sl] * wk[2]
                                    + rows_v[3, p, sl] * wk[3])
            return c2

        lax.fori_loop(0, P // 16, grp_body, 0)

        pltpu.sync_copy(out_v, out_hbm.at[pl.ds(off, P)])
        return carry

    lax.fori_loop(0, NCHUNK, chunk_body, 0)


@functools.cache
def _sc_gather_fn():
    return functools.partial(
        pl.kernel,
        mesh=plsc.VectorSubcoreMesh(
            core_axis_name="c", subcore_axis_name="s", num_cores=NC
        ),
        out_type=jax.ShapeDtypeStruct((B * NPIX, C), jnp.float32),
        compiler_params=pltpu.CompilerParams(use_tc_tiling_on_sc=False),
        scratch_types=[
            pltpu.VMEM((4, P), jnp.int32),
            pltpu.VMEM((4, P), jnp.float32),
            pltpu.VMEM((4, P, C), jnp.float32),
            pltpu.VMEM((P, C), jnp.float32),
            pltpu.SemaphoreType.DMA,
        ],
    )(_sc_body)


@jax.jit
def kernel(feat_rgb, rot, shift_u, shift_v):
    img_t = (
        feat_rgb[:, :, HALF:, :]
        .transpose(0, 2, 3, 1)
        .reshape(B * NROWS, C)
    )
    idx4, w4 = _compute_coords(rot, shift_u, shift_v)
    idx4 = idx4.reshape(4, B * NPIX)
    w4 = w4.reshape(4, B * NPIX)
    out = _sc_gather_fn()(img_t, idx4, w4)
    return out.reshape(B, BEV, BEV, C).transpose(0, 3, 1, 2)


# speedup vs baseline: 7.6518x; 1.1920x over previous
import functools
import math

import jax
import jax.numpy as jnp
from jax import lax
from jax.experimental import pallas as pl
from jax.experimental.pallas import tpu as pltpu
from jax.experimental.pallas import tpu_sc as plsc

BEV = 256
IH = 512
IW = 512
HALF = 256
C = 96
B = 2
NPIX = BEV * BEV
NROWS = (IH - HALF) * IW

NC = 2
NS = 16
NW = NC * NS
PER_W = B * NPIX // NW
P = 128
NCHUNK = PER_W // P

_TWO_PI = 2.0 * math.pi


def _coord_body(rot_ref, su_ref, sv_ref, idx_ref, w_ref):
    b = pl.program_id(0)
    i = lax.broadcasted_iota(jnp.int32, (BEV, BEV), 0).astype(jnp.float32)
    j = lax.broadcasted_iota(jnp.int32, (BEV, BEV), 1).astype(jnp.float32)
    cv = BEV / 2 - 0.5 + sv_ref[b]
    cu = BEV / 2 - 0.5 + su_ref[b]
    dy = i - cv
    dx = j - cu
    radius = jnp.sqrt(dy * dy + dx * dx)
    theta = jnp.arctan2(dy, dx)
    theta = (-math.pi / 2 + theta % _TWO_PI) % _TWO_PI
    theta = (theta + rot_ref[b] * _TWO_PI) % _TWO_PI
    u = theta / _TWO_PI * IW
    phi = jnp.arctan2(radius, jnp.float32(-2.0))
    v = phi / math.pi * IH

    fx = jnp.floor(u)
    fy = jnp.floor(v)
    ix0 = jnp.clip(fx, 0.0, IW - 1)
    ix1 = jnp.clip(fx + 1.0, 0.0, IW - 1)
    iy0 = jnp.clip(fy, 0.0, IH - 1)
    iy1 = jnp.clip(fy + 1.0, 0.0, IH - 1)

    w_ref[0, 0] = (ix1 - u) * (iy1 - v)
    w_ref[1, 0] = (u - ix0) * (iy1 - v)
    w_ref[2, 0] = (ix1 - u) * (v - iy0)
    w_ref[3, 0] = (u - ix0) * (v - iy0)

    base = b * NROWS
    ix0i = ix0.astype(jnp.int32)
    ix1i = ix1.astype(jnp.int32)
    ry0 = jnp.clip(iy0.astype(jnp.int32) - HALF, 0, IH - HALF - 1) * IW
    ry1 = jnp.clip(iy1.astype(jnp.int32) - HALF, 0, IH - HALF - 1) * IW
    idx_ref[0, 0] = base + ry0 + ix0i
    idx_ref[1, 0] = base + ry0 + ix1i
    idx_ref[2, 0] = base + ry1 + ix0i
    idx_ref[3, 0] = base + ry1 + ix1i


def _compute_coords(rot, shift_u, shift_v):
    return pl.pallas_call(
        _coord_body,
        grid=(B,),
        in_specs=[
            pl.BlockSpec(memory_space=pltpu.SMEM),
            pl.BlockSpec(memory_space=pltpu.SMEM),
            pl.BlockSpec(memory_space=pltpu.SMEM),
        ],
        out_specs=[
            pl.BlockSpec((4, 1, BEV, BEV), lambda b: (0, b, 0, 0)),
            pl.BlockSpec((4, 1, BEV, BEV), lambda b: (0, b, 0, 0)),
        ],
        out_shape=[
            jax.ShapeDtypeStruct((4, B, BEV, BEV), jnp.int32),
            jax.ShapeDtypeStruct((4, B, BEV, BEV), jnp.float32),
        ],
    )(rot, shift_u, shift_v)


def _sc_body(img_hbm, idx_hbm, w_hbm, out_hbm, idx_v, w_v, rows_v, out_v,
             sem0, sem1):
    wid = lax.axis_index("s") * NC + lax.axis_index("c")
    base = wid * PER_W
    sems = (sem0, sem1)

    def load_meta(t, slot):
        off = base + t * P
        pltpu.sync_copy(idx_hbm.at[:, pl.ds(off, P)], idx_v.at[slot])
        pltpu.sync_copy(w_hbm.at[:, pl.ds(off, P)], w_v.at[slot])

    def issue(slot):
        for k in range(4):
            pltpu.async_copy(
                img_hbm.at[idx_v.at[slot, k]], rows_v.at[slot, k], sems[slot]
            )

    def drain(slot):
        for k in range(4):
            pltpu.make_async_copy(
                img_hbm.at[pl.ds(0, P)], rows_v.at[slot, k], sems[slot]
            ).wait()

    def compute_store(t, slot):
        def grp_body(g, c2):
            p0 = g * 16
            wvec = [w_v[slot, k, pl.ds(p0, 16)] for k in range(4)]
            for pi in range(16):
                p = p0 + pi
                wk = [jnp.full((16,), wvec[k][pi], jnp.float32)
                      for k in range(4)]
                for vi in range(C // 16):
                    sl = pl.ds(vi * 16, 16)
                    out_v[p, sl] = (rows_v[slot, 0, p, sl] * wk[0]
                                    + rows_v[slot, 1, p, sl] * wk[1]
                                    + rows_v[slot, 2, p, sl] * wk[2]
                                    + rows_v[slot, 3, p, sl] * wk[3])
            return c2

        lax.fori_loop(0, P // 16, grp_body, 0)
        pltpu.sync_copy(out_v, out_hbm.at[pl.ds(base + t * P, P)])

    for b2 in range(2):
        load_meta(b2, b2)
        issue(b2)

    def ring_body(g, carry):
        for b2 in range(2):
            t = g * 2 + b2
            drain(b2)
            compute_store(t, b2)
            load_meta(t + 2, b2)
            issue(b2)
        return carry

    lax.fori_loop(0, NCHUNK // 2 - 1, ring_body, 0)

    for b2 in range(2):
        drain(b2)
        compute_store(NCHUNK - 2 + b2, b2)


@functools.cache
def _sc_gather_fn():
    return functools.partial(
        pl.kernel,
        mesh=plsc.VectorSubcoreMesh(
            core_axis_name="c", subcore_axis_name="s", num_cores=NC
        ),
        out_type=jax.ShapeDtypeStruct((B * NPIX, C), jnp.float32),
        compiler_params=pltpu.CompilerParams(use_tc_tiling_on_sc=False),
        scratch_types=[
            pltpu.VMEM((2, 4, P), jnp.int32),
            pltpu.VMEM((2, 4, P), jnp.float32),
            pltpu.VMEM((2, 4, P, C), jnp.float32),
            pltpu.VMEM((P, C), jnp.float32),
            pltpu.SemaphoreType.DMA,
            pltpu.SemaphoreType.DMA,
        ],
    )(_sc_body)


@jax.jit
def kernel(feat_rgb, rot, shift_u, shift_v):
    img_t = (
        feat_rgb[:, :, HALF:, :]
        .transpose(0, 2, 3, 1)
        .reshape(B * NROWS, C)
    )
    idx4, w4 = _compute_coords(rot, shift_u, shift_v)
    idx4 = idx4.reshape(4, B * NPIX)
    w4 = w4.reshape(4, B * NPIX)
    out = _sc_gather_fn()(img_t, idx4, w4)
    return out.reshape(B, BEV, BEV, C).transpose(0, 3, 1, 2)


# speedup vs baseline: 7.7063x; 1.0071x over previous
import functools
import math

import jax
import jax.numpy as jnp
from jax import lax
from jax.experimental import pallas as pl
from jax.experimental.pallas import tpu as pltpu
from jax.experimental.pallas import tpu_sc as plsc

BEV = 256
IH = 512
IW = 512
HALF = 256
C = 96
B = 2
NPIX = BEV * BEV
NROWS = (IH - HALF) * IW

NC = 2
NS = 16
NW = NC * NS
PER_W = B * NPIX // NW
P = 128
NCHUNK = PER_W // P

_TWO_PI = 2.0 * math.pi


def _coord_body(rot_ref, su_ref, sv_ref, idx_ref, w_ref):
    b = pl.program_id(0)
    i = lax.broadcasted_iota(jnp.int32, (BEV, BEV), 0).astype(jnp.float32)
    j = lax.broadcasted_iota(jnp.int32, (BEV, BEV), 1).astype(jnp.float32)
    cv = BEV / 2 - 0.5 + sv_ref[b]
    cu = BEV / 2 - 0.5 + su_ref[b]
    dy = i - cv
    dx = j - cu
    radius = jnp.sqrt(dy * dy + dx * dx)
    theta = jnp.arctan2(dy, dx)
    theta = (-math.pi / 2 + theta % _TWO_PI) % _TWO_PI
    theta = (theta + rot_ref[b] * _TWO_PI) % _TWO_PI
    u = theta / _TWO_PI * IW
    phi = jnp.arctan2(radius, jnp.float32(-2.0))
    v = phi / math.pi * IH

    fx = jnp.floor(u)
    fy = jnp.floor(v)
    ix0 = jnp.clip(fx, 0.0, IW - 1)
    ix1 = jnp.clip(fx + 1.0, 0.0, IW - 1)
    iy0 = jnp.clip(fy, 0.0, IH - 1)
    iy1 = jnp.clip(fy + 1.0, 0.0, IH - 1)

    w_ref[0, 0] = (ix1 - u) * (iy1 - v)
    w_ref[1, 0] = (u - ix0) * (iy1 - v)
    w_ref[2, 0] = (ix1 - u) * (v - iy0)
    w_ref[3, 0] = (u - ix0) * (v - iy0)

    base = b * NROWS
    ix0i = ix0.astype(jnp.int32)
    ix1i = ix1.astype(jnp.int32)
    ry0 = jnp.clip(iy0.astype(jnp.int32) - HALF, 0, IH - HALF - 1) * IW
    ry1 = jnp.clip(iy1.astype(jnp.int32) - HALF, 0, IH - HALF - 1) * IW
    idx_ref[0, 0] = base + ry0 + ix0i
    idx_ref[1, 0] = base + ry0 + ix1i
    idx_ref[2, 0] = base + ry1 + ix0i
    idx_ref[3, 0] = base + ry1 + ix1i


def _compute_coords(rot, shift_u, shift_v):
    return pl.pallas_call(
        _coord_body,
        grid=(B,),
        in_specs=[
            pl.BlockSpec(memory_space=pltpu.SMEM),
            pl.BlockSpec(memory_space=pltpu.SMEM),
            pl.BlockSpec(memory_space=pltpu.SMEM),
        ],
        out_specs=[
            pl.BlockSpec((4, 1, BEV, BEV), lambda b: (0, b, 0, 0)),
            pl.BlockSpec((4, 1, BEV, BEV), lambda b: (0, b, 0, 0)),
        ],
        out_shape=[
            jax.ShapeDtypeStruct((4, B, BEV, BEV), jnp.int32),
            jax.ShapeDtypeStruct((4, B, BEV, BEV), jnp.float32),
        ],
    )(rot, shift_u, shift_v)


RB = 8
TK = 4096


def _tin_body(src_ref, dst_ref):
    dst_ref[...] = src_ref[0].reshape(C, RB * IW).T


def _transpose_in(feat):
    return pl.pallas_call(
        _tin_body,
        grid=(B, (IH - HALF) // RB),
        in_specs=[
            pl.BlockSpec((1, C, RB, IW), lambda b, r: (b, 0, HALF // RB + r, 0))
        ],
        out_specs=pl.BlockSpec(
            (RB * IW, C), lambda b, r: (b * (NROWS // (RB * IW)) + r, 0)
        ),
        out_shape=jax.ShapeDtypeStruct((B * NROWS, C), jnp.float32),
    )(feat)


def _tout_body(src_ref, dst_ref):
    dst_ref[0] = src_ref[...].T


def _transpose_out(out_pm):
    return pl.pallas_call(
        _tout_body,
        grid=(B, NPIX // TK),
        in_specs=[
            pl.BlockSpec((TK, C), lambda b, r: (b * (NPIX // TK) + r, 0))
        ],
        out_specs=pl.BlockSpec((1, C, TK), lambda b, r: (b, 0, r)),
        out_shape=jax.ShapeDtypeStruct((B, C, NPIX), jnp.float32),
    )(out_pm)


def _sc_body(img_hbm, idx_hbm, w_hbm, out_hbm, idx_v, w_v, rows_v, out_v,
             sem0, sem1):
    wid = lax.axis_index("s") * NC + lax.axis_index("c")
    base = wid * PER_W
    sems = (sem0, sem1)

    def load_meta(t, slot):
        off = base + t * P
        pltpu.sync_copy(idx_hbm.at[:, pl.ds(off, P)], idx_v.at[slot])
        pltpu.sync_copy(w_hbm.at[:, pl.ds(off, P)], w_v.at[slot])

    def issue(slot):
        for k in range(4):
            pltpu.async_copy(
                img_hbm.at[idx_v.at[slot, k]], rows_v.at[slot, k], sems[slot]
            )

    def drain(slot):
        for k in range(4):
            pltpu.make_async_copy(
                img_hbm.at[pl.ds(0, P)], rows_v.at[slot, k], sems[slot]
            ).wait()

    def compute_store(t, slot):
        def grp_body(g, c2):
            p0 = g * 16
            wvec = [w_v[slot, k, pl.ds(p0, 16)] for k in range(4)]
            for pi in range(16):
                p = p0 + pi
                wk = [jnp.full((16,), wvec[k][pi], jnp.float32)
                      for k in range(4)]
                for vi in range(C // 16):
                    sl = pl.ds(vi * 16, 16)
                    out_v[p, sl] = (rows_v[slot, 0, p, sl] * wk[0]
                                    + rows_v[slot, 1, p, sl] * wk[1]
                                    + rows_v[slot, 2, p, sl] * wk[2]
                                    + rows_v[slot, 3, p, sl] * wk[3])
            return c2

        lax.fori_loop(0, P // 16, grp_body, 0)
        pltpu.sync_copy(out_v, out_hbm.at[pl.ds(base + t * P, P)])

    for b2 in range(2):
        load_meta(b2, b2)
        issue(b2)

    def ring_body(g, carry):
        for b2 in range(2):
            t = g * 2 + b2
            drain(b2)
            compute_store(t, b2)
            load_meta(t + 2, b2)
            issue(b2)
        return carry

    lax.fori_loop(0, NCHUNK // 2 - 1, ring_body, 0)

    for b2 in range(2):
        drain(b2)
        compute_store(NCHUNK - 2 + b2, b2)


@functools.cache
def _sc_gather_fn():
    return functools.partial(
        pl.kernel,
        mesh=plsc.VectorSubcoreMesh(
            core_axis_name="c", subcore_axis_name="s", num_cores=NC
        ),
        out_type=jax.ShapeDtypeStruct((B * NPIX, C), jnp.float32),
        compiler_params=pltpu.CompilerParams(use_tc_tiling_on_sc=False),
        scratch_types=[
            pltpu.VMEM((2, 4, P), jnp.int32),
            pltpu.VMEM((2, 4, P), jnp.float32),
            pltpu.VMEM((2, 4, P, C), jnp.float32),
            pltpu.VMEM((P, C), jnp.float32),
            pltpu.SemaphoreType.DMA,
            pltpu.SemaphoreType.DMA,
        ],
    )(_sc_body)


@jax.jit
def kernel(feat_rgb, rot, shift_u, shift_v):
    img_t = _transpose_in(feat_rgb)
    idx4, w4 = _compute_coords(rot, shift_u, shift_v)
    idx4 = idx4.reshape(4, B * NPIX)
    w4 = w4.reshape(4, B * NPIX)
    out = _sc_gather_fn()(img_t, idx4, w4)
    return _transpose_out(out).reshape(B, C, BEV, BEV)


# speedup vs baseline: 10.6635x; 1.3837x over previous
import functools
import math

import jax
import jax.numpy as jnp
from jax import lax
from jax.experimental import pallas as pl
from jax.experimental.pallas import tpu as pltpu
from jax.experimental.pallas import tpu_sc as plsc

BEV = 256
IH = 512
IW = 512
HALF = 256
C = 96
B = 2
NPIX = BEV * BEV
NROWS = (IH - HALF) * IW

NC = 2
NS = 16
NW = NC * NS
PER_W = B * NPIX // NW
P = 64
NCHUNK = PER_W // P
CP = 128

_TWO_PI = 2.0 * math.pi


def _coord_body(rot_ref, su_ref, sv_ref, idx_ref, w_ref):
    b = pl.program_id(0)
    i = lax.broadcasted_iota(jnp.int32, (BEV, BEV), 0).astype(jnp.float32)
    j = lax.broadcasted_iota(jnp.int32, (BEV, BEV), 1).astype(jnp.float32)
    cv = BEV / 2 - 0.5 + sv_ref[b]
    cu = BEV / 2 - 0.5 + su_ref[b]
    dy = i - cv
    dx = j - cu
    radius = jnp.sqrt(dy * dy + dx * dx)
    theta = jnp.arctan2(dy, dx)
    theta = (-math.pi / 2 + theta % _TWO_PI) % _TWO_PI
    theta = (theta + rot_ref[b] * _TWO_PI) % _TWO_PI
    u = theta / _TWO_PI * IW
    phi = jnp.arctan2(radius, jnp.float32(-2.0))
    v = phi / math.pi * IH

    fx = jnp.floor(u)
    fy = jnp.floor(v)
    ix0 = jnp.clip(fx, 0.0, IW - 1)
    ix1 = jnp.clip(fx + 1.0, 0.0, IW - 1)
    iy0 = jnp.clip(fy, 0.0, IH - 1)
    iy1 = jnp.clip(fy + 1.0, 0.0, IH - 1)

    w_ref[0, 0] = (ix1 - u) * (iy1 - v)
    w_ref[1, 0] = (u - ix0) * (iy1 - v)
    w_ref[2, 0] = (ix1 - u) * (v - iy0)
    w_ref[3, 0] = (u - ix0) * (v - iy0)

    base = b * NROWS
    ix0i = ix0.astype(jnp.int32)
    ix1i = ix1.astype(jnp.int32)
    ry0 = jnp.clip(iy0.astype(jnp.int32) - HALF, 0, IH - HALF - 1) * IW
    ry1 = jnp.clip(iy1.astype(jnp.int32) - HALF, 0, IH - HALF - 1) * IW
    idx_ref[0, 0] = base + ry0 + ix0i
    idx_ref[1, 0] = base + ry0 + ix1i
    idx_ref[2, 0] = base + ry1 + ix0i
    idx_ref[3, 0] = base + ry1 + ix1i


def _compute_coords(rot, shift_u, shift_v):
    return pl.pallas_call(
        _coord_body,
        grid=(B,),
        in_specs=[
            pl.BlockSpec(memory_space=pltpu.SMEM),
            pl.BlockSpec(memory_space=pltpu.SMEM),
            pl.BlockSpec(memory_space=pltpu.SMEM),
        ],
        out_specs=[
            pl.BlockSpec((4, 1, BEV, BEV), lambda b: (0, b, 0, 0)),
            pl.BlockSpec((4, 1, BEV, BEV), lambda b: (0, b, 0, 0)),
        ],
        out_shape=[
            jax.ShapeDtypeStruct((4, B, BEV, BEV), jnp.int32),
            jax.ShapeDtypeStruct((4, B, BEV, BEV), jnp.float32),
        ],
    )(rot, shift_u, shift_v)


RB = 8
TK = 4096


def _tin_body(src_ref, dst_ref):
    for j in range(RB):
        dst_ref[pl.ds(j * IW, IW), :C] = src_ref[0, :, j, :].T


def _transpose_in(feat):
    return pl.pallas_call(
        _tin_body,
        grid=(B, (IH - HALF) // RB),
        in_specs=[
            pl.BlockSpec((1, C, RB, IW), lambda b, r: (b, 0, HALF // RB + r, 0))
        ],
        out_specs=pl.BlockSpec(
            (RB * IW, CP), lambda b, r: (b * (NROWS // (RB * IW)) + r, 0)
        ),
        out_shape=jax.ShapeDtypeStruct((B * NROWS, CP), jnp.float32),
    )(feat)


def _tout_body(src_ref, dst_ref):
    for j in range(TK // BEV):
        dst_ref[0, :, j, :] = src_ref[pl.ds(j * BEV, BEV), :C].T


def _transpose_out(out_pm):
    return pl.pallas_call(
        _tout_body,
        grid=(B, NPIX // TK),
        in_specs=[
            pl.BlockSpec((TK, CP), lambda b, r: (b * (NPIX // TK) + r, 0))
        ],
        out_specs=pl.BlockSpec(
            (1, C, TK // BEV, BEV), lambda b, r: (b, 0, r, 0)
        ),
        out_shape=jax.ShapeDtypeStruct((B, C, BEV, BEV), jnp.float32),
    )(out_pm)


def _sc_body(img_hbm, idx_hbm, w_hbm, out_hbm, idx_v, w_v, rows_v, out_v,
             sem0, sem1):
    wid = lax.axis_index("s") * NC + lax.axis_index("c")
    base = wid * PER_W
    sems = (sem0, sem1)

    def load_meta(t, slot):
        off = base + t * P
        pltpu.sync_copy(idx_hbm.at[:, pl.ds(off, P)], idx_v.at[slot])
        pltpu.sync_copy(w_hbm.at[:, pl.ds(off, P)], w_v.at[slot])

    def issue(slot):
        for k in range(4):
            pltpu.async_copy(
                img_hbm.at[idx_v.at[slot, k]], rows_v.at[slot, k], sems[slot]
            )

    def drain(slot):
        for k in range(4):
            pltpu.make_async_copy(
                img_hbm.at[pl.ds(0, P)], rows_v.at[slot, k], sems[slot]
            ).wait()


    def compute_store(t, slot):
        def grp_body(g, c2):
            p0 = g * 16
            wvec = [w_v[slot, k, pl.ds(p0, 16)] for k in range(4)]
            for pi in range(16):
                p = p0 + pi
                wk = [jnp.full((16,), wvec[k][pi], jnp.float32)
                      for k in range(4)]
                for vi in range(C // 16):
                    sl = pl.ds(vi * 16, 16)
                    out_v[p, sl] = (rows_v[slot, 0, p, sl] * wk[0]
                                    + rows_v[slot, 1, p, sl] * wk[1]
                                    + rows_v[slot, 2, p, sl] * wk[2]
                                    + rows_v[slot, 3, p, sl] * wk[3])
            return c2

        lax.fori_loop(0, P // 16, grp_body, 0)
        pltpu.sync_copy(out_v, out_hbm.at[pl.ds(base + t * P, P)])

    for b2 in range(2):
        load_meta(b2, b2)
        issue(b2)

    def ring_body(g, carry):
        for b2 in range(2):
            t = g * 2 + b2
            drain(b2)
            compute_store(t, b2)
            load_meta(t + 2, b2)
            issue(b2)
        return carry

    lax.fori_loop(0, NCHUNK // 2 - 1, ring_body, 0)

    for b2 in range(2):
        drain(b2)
        compute_store(NCHUNK - 2 + b2, b2)


@functools.cache
def _sc_gather_fn():
    return functools.partial(
        pl.kernel,
        mesh=plsc.VectorSubcoreMesh(
            core_axis_name="c", subcore_axis_name="s", num_cores=NC
        ),
        out_type=jax.ShapeDtypeStruct((B * NPIX, CP), jnp.float32),
        compiler_params=pltpu.CompilerParams(use_tc_tiling_on_sc=False),
        scratch_types=[
            pltpu.VMEM((2, 4, P), jnp.int32),
            pltpu.VMEM((2, 4, P), jnp.float32),
            pltpu.VMEM((2, 4, P, CP), jnp.float32),
            pltpu.VMEM((P, CP), jnp.float32),
            pltpu.SemaphoreType.DMA,
            pltpu.SemaphoreType.DMA,
        ],
    )(_sc_body)


@jax.jit
def kernel(feat_rgb, rot, shift_u, shift_v):
    img_t = _transpose_in(feat_rgb)
    idx4, w4 = _compute_coords(rot, shift_u, shift_v)
    idx4 = idx4.reshape(4, B * NPIX)
    w4 = w4.reshape(4, B * NPIX)
    out = _sc_gather_fn()(img_t, idx4, w4)
    return _transpose_out(out)
